# baseline (speedup 1.0000x reference)
import functools

import jax
import jax.numpy as jnp
from jax import lax
from jax.experimental import pallas as pl
from jax.experimental.pallas import tpu as pltpu
from jax.experimental.pallas import tpu_sc as plsc

N = 10000
NP = 10240
E = 160000
H = 512
NLAYERS = 3
NSLAB = 4
CS = 128
NC, NS = 2, 16
NW = NC * NS
EPAD = 165888
EW_T = EPAD // NW
ECH_T = EPAD // NS
NCH = ECH_T // 128
ACC_ROWS = 4096
RB = 1024

_mesh = plsc.VectorSubcoreMesh(core_axis_name="c", subcore_axis_name="s")
_sc_params = pltpu.CompilerParams(needs_layout_passes=False)


def _deg_body(col_hbm, ew_hbm, out_hbm, colv, ewv, degv):
    c = lax.axis_index("c")
    s = lax.axis_index("s")
    wid = s * NC + c
    base = wid * EW_T
    pltpu.sync_copy(col_hbm.at[pl.ds(base, EW_T)], colv)
    pltpu.sync_copy(ew_hbm.at[pl.ds(base, EW_T)], ewv)

    def zbody(i, carry):
        degv[pl.ds(i * 16, 16)] = jnp.zeros((16,), jnp.float32)
        return carry

    lax.fori_loop(0, NP // 16, zbody, 0)

    def ebody(i, carry):
        c16 = colv[pl.ds(i * 16, 16)]
        w16 = ewv[pl.ds(i * 16, 16)]
        plsc.addupdate_scatter(degv, [c16], w16)
        return carry

    lax.fori_loop(0, EW_T // 16, ebody, 0)
    pltpu.sync_copy(degv, out_hbm.at[pl.ds(wid * NP, NP)])


_deg_call = functools.partial(
    pl.kernel,
    out_type=jax.ShapeDtypeStruct((NW * NP,), jnp.float32),
    mesh=_mesh,
    scratch_types=[
        pltpu.VMEM((EW_T,), jnp.int32),
        pltpu.VMEM((EW_T,), jnp.float32),
        pltpu.VMEM((NP,), jnp.float32),
    ],
    compiler_params=_sc_params,
)(_deg_body)


def _msg_body(hw_hbm, row_hbm, col_hbm, ew_hbm, z_hbm, out_hbm,
              rowv, colv, ewv, widx, gbuf, zbuf, acc, sem0, sem1, sem2):
    c = lax.axis_index("c")
    s = lax.axis_index("s")
    pltpu.sync_copy(row_hbm.at[s], rowv)
    pltpu.sync_copy(col_hbm.at[s], colv)
    pltpu.sync_copy(ew_hbm.at[s], ewv)
    pltpu.sync_copy(z_hbm, zbuf)
    sems = (sem0, sem1, sem2)
    stripe = ACC_ROWS // NS

    def do_round(d, base_r, rows_w):
        for z in range(stripe // 64):
            pltpu.sync_copy(zbuf, acc.at[pl.ds(s * stripe + z * 64, 64)])
        plsc.subcore_barrier()

        def gissue(j, b):
            pltpu.async_copy(hw_hbm.at[d].at[rowv.at[j]], gbuf.at[b],
                             sems[b])

        def gwait(j, b):
            pltpu.make_async_copy(hw_hbm.at[d].at[rowv.at[j]], gbuf.at[b],
                                  sems[b]).wait()

        def sissue(j, b):
            pltpu.async_copy(gbuf.at[b], acc.at[widx.at[b]], sems[b],
                             add=True)

        def swait(j, b):
            pltpu.make_async_copy(gbuf.at[b], acc.at[widx.at[b]],
                                  sems[b]).wait()

        gissue(0, 0)
        gissue(1, 1)
        gissue(2, 2)

        def ebody(ii, carry):
            for t in range(3):
                j = ii * 3 + t
                b = t
                gwait(j, b)

                def scale(g, carry2):
                    sl16 = pl.ds(g * 16, 16)
                    c16 = colv[j, sl16]
                    loc16 = c16 - base_r
                    m = (loc16 >= 0) & (loc16 < ACC_ROWS)
                    widx[b, sl16] = jnp.where(m, loc16, 0)
                    w16 = jnp.where(m, ewv[j, sl16], 0.0)
                    for l in range(16):
                        w = w16[l]
                        e = g * 16 + l
                        for c8 in range(8):
                            sl = pl.ds(c8 * 16, 16)
                            gbuf[b, e, sl] = gbuf[b, e, sl] * w
                    return carry2

                lax.fori_loop(0, 8, scale, 0)
                sissue(j, b)
                bb = (t + 2) % 3

                @pl.when((j >= 1) & (j + 2 < NCH))
                def _():
                    swait(j - 1, bb)
                    gissue(j + 2, bb)
            return carry

        lax.fori_loop(0, NCH // 3, ebody, 0)
        swait(NCH - 3, (NCH - 3) % 3)
        swait(NCH - 2, (NCH - 2) % 3)
        swait(NCH - 1, (NCH - 1) % 3)
        plsc.subcore_barrier()
        stripe_w = rows_w // NS
        for z in range(stripe_w // 64):
            off = s * stripe_w + z * 64
            pltpu.sync_copy(acc.at[pl.ds(off, 64)],
                            out_hbm.at[d].at[pl.ds(base_r + off, 64)])
        plsc.subcore_barrier()

    for cc in range(NC):
        @pl.when(c == cc)
        def _():
            for k in range(2):
                d = cc * 2 + k
                do_round(d, 0, ACC_ROWS)
                do_round(d, ACC_ROWS, ACC_ROWS)
                do_round(d, 2 * ACC_ROWS, NP - 2 * ACC_ROWS)


_msg_call = functools.partial(
    pl.kernel,
    out_type=jax.ShapeDtypeStruct((NSLAB, NP, CS), jnp.float32),
    mesh=_mesh,
    scratch_types=[
        pltpu.VMEM((NCH, 128), jnp.int32),
        pltpu.VMEM((NCH, 128), jnp.int32),
        pltpu.VMEM((NCH, 128), jnp.float32),
        pltpu.VMEM((3, 128), jnp.int32),
        pltpu.VMEM((3, 128, CS), jnp.float32),
        pltpu.VMEM((64, CS), jnp.float32),
        pltpu.VMEM_SHARED((ACC_ROWS, CS), jnp.float32),
        pltpu.SemaphoreType.DMA,
        pltpu.SemaphoreType.DMA,
        pltpu.SemaphoreType.DMA,
    ],
    compiler_params=_sc_params,
)(_msg_body)


def _mm_in_body(x_ref, w_ref, b_ref, o_ref):
    o_ref[...] = (
        jnp.dot(x_ref[...], w_ref[...], preferred_element_type=jnp.float32)
        + b_ref[...]
    )


def _dinv_body(p_ref, o_ref):
    deg = 1.0 + jnp.sum(p_ref[...], axis=0, keepdims=True)
    o_ref[...] = lax.rsqrt(jnp.maximum(deg, 1e-12))


def _t1_body(h_ref, w_ref, dinv_ref, o_ref):
    acc = jnp.dot(h_ref[...], w_ref[...], preferred_element_type=jnp.float32)
    acc = acc * dinv_ref[...]
    for d in range(NSLAB):
        o_ref[d] = acc[:, d * CS:(d + 1) * CS]


def _t2_body(m_ref, hw_ref, dinv_ref, b_ref, g_ref, bb_ref, o_ref):
    m = jnp.concatenate([m_ref[d] for d in range(NSLAB)], axis=1)
    hw = jnp.concatenate([hw_ref[d] for d in range(NSLAB)], axis=1)
    di = dinv_ref[...]
    g = di * (m + hw) + b_ref[...]
    mu = jnp.mean(g, axis=1, keepdims=True)
    var = jnp.mean((g - mu) * (g - mu), axis=1, keepdims=True)
    y = (g - mu) * lax.rsqrt(var + 1e-5) * g_ref[...] + bb_ref[...]
    o_ref[...] = jnp.maximum(y, 0.0)


def _fc_body(h_ref, w_ref, b_ref, o_ref):
    o_ref[...] = (
        jnp.dot(h_ref[...], w_ref[...], preferred_element_type=jnp.float32)
        + b_ref[...]
    )


def _mm_in(xp, Wp, b2):
    return pl.pallas_call(
        _mm_in_body,
        grid=(NP // RB,),
        in_specs=[
            pl.BlockSpec((RB, 8), lambda i: (i, 0)),
            pl.BlockSpec((8, H), lambda i: (0, 0)),
            pl.BlockSpec((1, H), lambda i: (0, 0)),
        ],
        out_specs=pl.BlockSpec((RB, H), lambda i: (i, 0)),
        out_shape=jax.ShapeDtypeStruct((NP, H), jnp.float32),
    )(xp, Wp, b2)


def _dinv(partials):
    return pl.pallas_call(
        _dinv_body,
        in_specs=[pl.BlockSpec((NW, NP), lambda: (0, 0))],
        out_specs=pl.BlockSpec((1, NP), lambda: (0, 0)),
        out_shape=jax.ShapeDtypeStruct((1, NP), jnp.float32),
    )(partials)


def _t1(h, W, dinv_c):
    return pl.pallas_call(
        _t1_body,
        grid=(NP // RB,),
        in_specs=[
            pl.BlockSpec((RB, H), lambda i: (i, 0)),
            pl.BlockSpec((H, H), lambda i: (0, 0)),
            pl.BlockSpec((RB, 1), lambda i: (i, 0)),
        ],
        out_specs=pl.BlockSpec((NSLAB, RB, CS), lambda i: (0, i, 0)),
        out_shape=jax.ShapeDtypeStruct((NSLAB, NP, CS), jnp.float32),
    )(h, W, dinv_c)


def _t2(msg, hw, dinv_c, b2, g2, bb2):
    return pl.pallas_call(
        _t2_body,
        grid=(NP // RB,),
        in_specs=[
            pl.BlockSpec((NSLAB, RB, CS), lambda i: (0, i, 0)),
            pl.BlockSpec((NSLAB, RB, CS), lambda i: (0, i, 0)),
            pl.BlockSpec((RB, 1), lambda i: (i, 0)),
            pl.BlockSpec((1, H), lambda i: (0, 0)),
            pl.BlockSpec((1, H), lambda i: (0, 0)),
            pl.BlockSpec((1, H), lambda i: (0, 0)),
        ],
        out_specs=pl.BlockSpec((RB, H), lambda i: (i, 0)),
        out_shape=jax.ShapeDtypeStruct((NP, H), jnp.float32),
    )(msg, hw, dinv_c, b2, g2, bb2)


def _fc(h, W, b2):
    return pl.pallas_call(
        _fc_body,
        grid=(NP // RB,),
        in_specs=[
            pl.BlockSpec((RB, H), lambda i: (i, 0)),
            pl.BlockSpec((H, H), lambda i: (0, 0)),
            pl.BlockSpec((1, H), lambda i: (0, 0)),
        ],
        out_specs=pl.BlockSpec((RB, H), lambda i: (i, 0)),
        out_shape=jax.ShapeDtypeStruct((NP, H), jnp.float32),
    )(h, W, b2)


def kernel(x, edge_index, edge_attr, W_in, b_in, conv_W, conv_b, ln_g, ln_b,
           W_fc, b_fc):
    x2 = x[0]
    row = edge_index[0, 0]
    col = edge_index[0, 1]
    ew = edge_attr[0, :, 0]
    row_p = jnp.pad(row, (0, EPAD - E))
    col_p = jnp.pad(col, (0, EPAD - E))
    ew_p = jnp.pad(ew, (0, EPAD - E))
    row2 = row_p.reshape(NS, NCH, 128)
    col2 = col_p.reshape(NS, NCH, 128)
    ew2 = ew_p.reshape(NS, NCH, 128)
    zeros_z = jnp.zeros((64, CS), jnp.float32)

    xp = jnp.pad(x2, ((0, NP - N), (0, 5)))
    Wp = jnp.pad(W_in, ((0, 5), (0, 0)))

    partials = _deg_call(col_p, ew_p).reshape(NW, NP)
    h = _mm_in(xp, Wp, b_in.reshape(1, H))
    dinv_c = _dinv(partials).reshape(NP, 1)

    for i in range(NLAYERS):
        hw = _t1(h, conv_W[i], dinv_c)
        msg = _msg_call(hw, row2, col2, ew2, zeros_z)
        h = _t2(msg, hw, dinv_c, conv_b[i].reshape(1, H),
                ln_g[i].reshape(1, H), ln_b[i].reshape(1, H))

    return _fc(h, W_fc, b_fc.reshape(1, H))[:N]


# speedup vs baseline: 1.7286x; 1.7286x over previous
import functools

import jax
import jax.numpy as jnp
from jax import lax
from jax.experimental import pallas as pl
from jax.experimental.pallas import tpu as pltpu
from jax.experimental.pallas import tpu_sc as plsc

N = 10000
NP = 10240
E = 160000
H = 512
NLAYERS = 3
NSLAB = 4
CS = 128
NC, NS = 2, 16
NW = NC * NS
EPAD = 163840
EW_T = EPAD // NW
ECH_T = EPAD // NS
NCH = ECH_T // 128
ACC_ROWS = 6144
RB = 1024

_mesh = plsc.VectorSubcoreMesh(core_axis_name="c", subcore_axis_name="s")
_sc_params = pltpu.CompilerParams(needs_layout_passes=False)


def _deg_body(col_hbm, ew_hbm, out_hbm, colv, ewv, degv):
    c = lax.axis_index("c")
    s = lax.axis_index("s")
    wid = s * NC + c
    base = wid * EW_T
    pltpu.sync_copy(col_hbm.at[pl.ds(base, EW_T)], colv)
    pltpu.sync_copy(ew_hbm.at[pl.ds(base, EW_T)], ewv)

    def zbody(i, carry):
        degv[pl.ds(i * 16, 16)] = jnp.zeros((16,), jnp.float32)
        return carry

    lax.fori_loop(0, NP // 16, zbody, 0)

    def ebody(i, carry):
        c16 = colv[pl.ds(i * 16, 16)]
        w16 = ewv[pl.ds(i * 16, 16)]
        plsc.addupdate_scatter(degv, [c16], w16)
        return carry

    lax.fori_loop(0, EW_T // 16, ebody, 0)
    pltpu.sync_copy(degv, out_hbm.at[pl.ds(wid * NP, NP)])


_deg_call = functools.partial(
    pl.kernel,
    out_type=jax.ShapeDtypeStruct((NW * NP,), jnp.float32),
    mesh=_mesh,
    scratch_types=[
        pltpu.VMEM((EW_T,), jnp.int32),
        pltpu.VMEM((EW_T,), jnp.float32),
        pltpu.VMEM((NP,), jnp.float32),
    ],
    compiler_params=_sc_params,
)(_deg_body)


def _msg_body(hw_hbm, row_hbm, col_hbm, ew_hbm, z_hbm, out_hbm,
              rowv, colv, ewv, widx, gbuf, zbuf, acc, gsem0, gsem1):
    c = lax.axis_index("c")
    s = lax.axis_index("s")
    pltpu.sync_copy(row_hbm.at[pl.ds(s * NCH, NCH)], rowv)
    pltpu.sync_copy(col_hbm.at[pl.ds(s * NCH, NCH)], colv)
    pltpu.sync_copy(ew_hbm.at[pl.ds(s * NCH, NCH)], ewv)
    pltpu.sync_copy(z_hbm, zbuf)
    gsems = (gsem0, gsem1)

    def do_round(d, base_r, rows_r):
        stripe = rows_r // NS
        for z in range(stripe // 128):
            pltpu.sync_copy(zbuf, acc.at[pl.ds(s * stripe + z * 128, 128)])
        plsc.subcore_barrier()

        def gissue(j, b):
            pltpu.async_copy(hw_hbm.at[d].at[rowv.at[j]], gbuf.at[b],
                             gsems[b])

        def gwait(j, b):
            pltpu.make_async_copy(hw_hbm.at[d].at[rowv.at[j]], gbuf.at[b],
                                  gsems[b]).wait()

        gissue(0, 0)
        gissue(1, 1)

        def ebody(jj, carry):
            for b in range(2):
                j = jj * 2 + b
                gwait(j, b)

                def scale(g, carry2):
                    sl16 = pl.ds(g * 16, 16)
                    c16 = colv[j, sl16]
                    loc16 = c16 - base_r
                    m = (loc16 >= 0) & (loc16 < rows_r)
                    widx[0, sl16] = jnp.where(m, loc16, 0)
                    w16 = jnp.where(m, ewv[j, sl16], 0.0)
                    for l in range(16):
                        w = w16[l]
                        e = g * 16 + l
                        for c8 in range(8):
                            sl = pl.ds(c8 * 16, 16)
                            gbuf[b, e, sl] = gbuf[b, e, sl] * w
                    return carry2

                lax.fori_loop(0, 8, scale, 0)
                pltpu.sync_copy(gbuf.at[b], acc.at[widx.at[0]], add=True)

                @pl.when(j + 2 < NCH)
                def _():
                    gissue(j + 2, b)
            return carry

        lax.fori_loop(0, NCH // 2, ebody, 0)
        plsc.subcore_barrier()
        for z in range(stripe // 128):
            off = s * stripe + z * 128
            pltpu.sync_copy(acc.at[pl.ds(off, 128)],
                            out_hbm.at[d].at[pl.ds(base_r + off, 128)])
        plsc.subcore_barrier()

    for cc in range(NC):
        @pl.when(c == cc)
        def _():
            for k in range(2):
                d = cc * 2 + k
                do_round(d, 0, ACC_ROWS)
                do_round(d, ACC_ROWS, NP - ACC_ROWS)


_msg_call = functools.partial(
    pl.kernel,
    out_type=jax.ShapeDtypeStruct((NSLAB, NP, CS), jnp.float32),
    mesh=_mesh,
    scratch_types=[
        pltpu.VMEM((NCH, 128), jnp.int32),
        pltpu.VMEM((NCH, 128), jnp.int32),
        pltpu.VMEM((NCH, 128), jnp.float32),
        pltpu.VMEM((1, 128), jnp.int32),
        pltpu.VMEM((2, 128, CS), jnp.float32),
        pltpu.VMEM((128, CS), jnp.float32),
        pltpu.VMEM_SHARED((ACC_ROWS, CS), jnp.float32),
        pltpu.SemaphoreType.DMA,
        pltpu.SemaphoreType.DMA,
    ],
    compiler_params=_sc_params,
)(_msg_body)


def _mm_in_body(x_ref, w_ref, b_ref, o_ref):
    o_ref[...] = (
        jnp.dot(x_ref[...], w_ref[...], preferred_element_type=jnp.float32)
        + b_ref[...]
    )


def _dinv_body(p_ref, o_ref):
    deg = 1.0 + jnp.sum(p_ref[...], axis=0, keepdims=True)
    o_ref[...] = lax.rsqrt(jnp.maximum(deg, 1e-12))


def _t1_body(h_ref, w_ref, dinv_ref, o_ref):
    acc = jnp.dot(h_ref[...], w_ref[...], preferred_element_type=jnp.float32)
    acc = acc * dinv_ref[...]
    for d in range(NSLAB):
        o_ref[d] = acc[:, d * CS:(d + 1) * CS]


def _t2_body(m_ref, hw_ref, dinv_ref, b_ref, g_ref, bb_ref, o_ref):
    m = jnp.concatenate([m_ref[d] for d in range(NSLAB)], axis=1)
    hw = jnp.concatenate([hw_ref[d] for d in range(NSLAB)], axis=1)
    di = dinv_ref[...]
    g = di * (m + hw) + b_ref[...]
    mu = jnp.mean(g, axis=1, keepdims=True)
    var = jnp.mean((g - mu) * (g - mu), axis=1, keepdims=True)
    y = (g - mu) * lax.rsqrt(var + 1e-5) * g_ref[...] + bb_ref[...]
    o_ref[...] = jnp.maximum(y, 0.0)


def _fc_body(h_ref, w_ref, b_ref, o_ref):
    o_ref[...] = (
        jnp.dot(h_ref[...], w_ref[...], preferred_element_type=jnp.float32)
        + b_ref[...]
    )


def _mm_in(xp, Wp, b2):
    return pl.pallas_call(
        _mm_in_body,
        grid=(NP // RB,),
        in_specs=[
            pl.BlockSpec((RB, 8), lambda i: (i, 0)),
            pl.BlockSpec((8, H), lambda i: (0, 0)),
            pl.BlockSpec((1, H), lambda i: (0, 0)),
        ],
        out_specs=pl.BlockSpec((RB, H), lambda i: (i, 0)),
        out_shape=jax.ShapeDtypeStruct((NP, H), jnp.float32),
    )(xp, Wp, b2)


def _dinv(partials):
    return pl.pallas_call(
        _dinv_body,
        in_specs=[pl.BlockSpec((NW, NP), lambda: (0, 0))],
        out_specs=pl.BlockSpec((1, NP), lambda: (0, 0)),
        out_shape=jax.ShapeDtypeStruct((1, NP), jnp.float32),
    )(partials)


def _t1(h, W, dinv_c):
    return pl.pallas_call(
        _t1_body,
        grid=(NP // RB,),
        in_specs=[
            pl.BlockSpec((RB, H), lambda i: (i, 0)),
            pl.BlockSpec((H, H), lambda i: (0, 0)),
            pl.BlockSpec((RB, 1), lambda i: (i, 0)),
        ],
        out_specs=pl.BlockSpec((NSLAB, RB, CS), lambda i: (0, i, 0)),
        out_shape=jax.ShapeDtypeStruct((NSLAB, NP, CS), jnp.float32),
    )(h, W, dinv_c)


def _t2(msg, hw, dinv_c, b2, g2, bb2):
    return pl.pallas_call(
        _t2_body,
        grid=(NP // RB,),
        in_specs=[
            pl.BlockSpec((NSLAB, RB, CS), lambda i: (0, i, 0)),
            pl.BlockSpec((NSLAB, RB, CS), lambda i: (0, i, 0)),
            pl.BlockSpec((RB, 1), lambda i: (i, 0)),
            pl.BlockSpec((1, H), lambda i: (0, 0)),
            pl.BlockSpec((1, H), lambda i: (0, 0)),
            pl.BlockSpec((1, H), lambda i: (0, 0)),
        ],
        out_specs=pl.BlockSpec((RB, H), lambda i: (i, 0)),
        out_shape=jax.ShapeDtypeStruct((NP, H), jnp.float32),
    )(msg, hw, dinv_c, b2, g2, bb2)


def _fc(h, W, b2):
    return pl.pallas_call(
        _fc_body,
        grid=(NP // RB,),
        in_specs=[
            pl.BlockSpec((RB, H), lambda i: (i, 0)),
            pl.BlockSpec((H, H), lambda i: (0, 0)),
            pl.BlockSpec((1, H), lambda i: (0, 0)),
        ],
        out_specs=pl.BlockSpec((RB, H), lambda i: (i, 0)),
        out_shape=jax.ShapeDtypeStruct((NP, H), jnp.float32),
    )(h, W, b2)


def kernel(x, edge_index, edge_attr, W_in, b_in, conv_W, conv_b, ln_g, ln_b,
           W_fc, b_fc):
    x2 = x[0]
    row = edge_index[0, 0]
    col = edge_index[0, 1]
    ew = edge_attr[0, :, 0]
    row_p = jnp.pad(row, (0, EPAD - E))
    col_p = jnp.pad(col, (0, EPAD - E))
    ew_p = jnp.pad(ew, (0, EPAD - E))
    row2 = row_p.reshape(-1, 128)
    col2 = col_p.reshape(-1, 128)
    ew2 = ew_p.reshape(-1, 128)
    zeros_z = jnp.zeros((128, CS), jnp.float32)

    xp = jnp.pad(x2, ((0, NP - N), (0, 5)))
    Wp = jnp.pad(W_in, ((0, 5), (0, 0)))

    partials = _deg_call(col_p, ew_p).reshape(NW, NP)
    h = _mm_in(xp, Wp, b_in.reshape(1, H))
    dinv_c = _dinv(partials).reshape(NP, 1)

    for i in range(NLAYERS):
        hw = _t1(h, conv_W[i], dinv_c)
        msg = _msg_call(hw, row2, col2, ew2, zeros_z)
        h = _t2(msg, hw, dinv_c, conv_b[i].reshape(1, H),
                ln_g[i].reshape(1, H), ln_b[i].reshape(1, H))

    return _fc(h, W_fc, b_fc.reshape(1, H))[:N]
